# Initial kernel scaffold; baseline (speedup 1.0000x reference)
import jax
import jax.numpy as jnp
from jax.experimental import pallas as pl


def kernel(input, dialog_lengths, speakers, W_rel, W_root, b_rgcn, Wgc_root, Wgc_rel, b_gc, W_skip, b_skip, W_cls, b_cls):
    raise NotImplementedError("write your pallas kernel here")


# speedup vs baseline: 3.9285x; 3.9285x over previous
import functools

import jax
import jax.numpy as jnp
from jax.experimental import pallas as pl
from jax.experimental.pallas import tpu as pltpu

_M = 1024


def _body(in_ref, spk_ref, pos_ref, w0_ref, w1_ref, wroot_ref, brg_ref,
          wgroot_ref, wgrel_ref, bgc_ref, wsk_ref, bsk_ref, wc_ref, bc_ref,
          out_ref):
    M = _M
    b = pl.program_id(0)
    base = b * M
    f32 = jnp.float32
    xin = in_ref[pl.ds(base, M + 8), :]
    spk = spk_ref[pl.ds(base, M + 8), :]
    pos = pos_ref[pl.ds(base, M + 8), :]

    h0 = jnp.dot(xin, w0_ref[...], preferred_element_type=f32)
    h1 = jnp.dot(xin, w1_ref[...], preferred_element_type=f32)
    root = jnp.dot(xin, wroot_ref[...], preferred_element_type=f32) + brg_ref[...]

    S = spk[4:, :]
    posx = pos[4:, :]
    valid = [(posx >= float(k)).astype(f32) for k in range(5)]
    sp_sh = [spk[4 - k:M + 8 - k, :] for k in range(5)]
    nv = valid[0] + valid[1] + valid[2] + valid[3] + valid[4]
    c1 = sum(v * s for v, s in zip(valid, sp_sh))
    c0 = nv - c1
    inv_nv = 1.0 / nv
    inv_c0 = 1.0 / jnp.maximum(c0, 1.0)
    inv_c1 = 1.0 / jnp.maximum(c1, 1.0)
    x = root[4:, :]
    for k in range(5):
        co0 = valid[k] * ((1.0 - S) * inv_nv + S * (1.0 - sp_sh[k]) * inv_c0)
        co1 = valid[k] * S * sp_sh[k] * inv_c1
        x = x + co0 * h0[4 - k:M + 8 - k, :] + co1 * h1[4 - k:M + 8 - k, :]

    posn = pos[8:, :]
    agg = x[4:, :]
    for k in range(1, 5):
        agg = agg + (posn >= float(k)).astype(f32) * x[4 - k:M + 4 - k, :]

    xo = x[4:, :]
    x2 = (jnp.dot(agg, wgrel_ref[...], preferred_element_type=f32)
          + jnp.dot(xo, wgroot_ref[...], preferred_element_type=f32)
          + bgc_ref[...])
    skip = jnp.dot(xin[8:, :], wsk_ref[...], preferred_element_type=f32) + bsk_ref[...]
    out_ref[...] = jnp.dot(x2 + skip, wc_ref[...], preferred_element_type=f32) + bc_ref[...]


def kernel(input, dialog_lengths, speakers, W_rel, W_root, b_rgcn,
           Wgc_root, Wgc_rel, b_gc, W_skip, b_skip, W_cls, b_cls):
    N, D = input.shape
    n_cls = W_cls.shape[1]
    M = _M
    nblocks = (N + M - 1) // M
    Npad = nblocks * M
    tot = Npad + 8

    ends = jnp.cumsum(dialog_lengths)
    offsets = ends - dialog_lengths
    d_idx = jnp.searchsorted(ends, jnp.arange(N), side='right')
    pos = (jnp.arange(N) - offsets[d_idx]).astype(jnp.float32)

    inp_p = jnp.zeros((tot, D), input.dtype).at[8:8 + N].set(input)
    spk_p = jnp.zeros((tot, 1), jnp.float32).at[8:8 + N, 0].set(
        speakers.astype(jnp.float32))
    pos_p = jnp.zeros((tot, 1), jnp.float32).at[8:8 + N, 0].set(pos)

    full = lambda shape: pl.BlockSpec(shape, lambda b: (0,) * len(shape))
    out = pl.pallas_call(
        _body,
        grid=(nblocks,),
        in_specs=[
            full((tot, D)), full((tot, 1)), full((tot, 1)),
            full((D, D)), full((D, D)), full((D, D)), full((1, D)),
            full((D, D)), full((D, D)), full((1, D)),
            full((D, D)), full((1, D)),
            full((D, n_cls)), full((1, n_cls)),
        ],
        out_specs=pl.BlockSpec((M, n_cls), lambda b: (b, 0)),
        out_shape=jax.ShapeDtypeStruct((Npad, n_cls), jnp.float32),
    )(inp_p, spk_p, pos_p, W_rel[0], W_rel[1], W_root,
      b_rgcn.reshape(1, D), Wgc_root, Wgc_rel, b_gc.reshape(1, D),
      W_skip, b_skip.reshape(1, D), W_cls, b_cls.reshape(1, n_cls))
    return out[:N]


# speedup vs baseline: 33.4632x; 8.5181x over previous
import functools

import jax
import jax.numpy as jnp
from jax.experimental import pallas as pl
from jax.experimental.pallas import tpu as pltpu

_M = 1024


def _body(in_ref, spk_ref, pos_ref, w0_ref, w1_ref, wroot_ref, brg_ref,
          wgroot_ref, wgrel_ref, bgc_ref, wsk_ref, bsk_ref, wc_ref, bc_ref,
          out_ref):
    M = _M
    b = pl.program_id(0)
    base = b * M
    f32 = jnp.float32
    xin = in_ref[pl.ds(base, M + 8), :]
    spk = spk_ref[pl.ds(base, M + 8), :]
    pos = pos_ref[pl.ds(base, M + 8), :]

    h0 = jnp.dot(xin, w0_ref[...], preferred_element_type=f32)
    h1 = jnp.dot(xin, w1_ref[...], preferred_element_type=f32)
    root = jnp.dot(xin, wroot_ref[...], preferred_element_type=f32) + brg_ref[...]

    S = spk[4:, :]
    posx = pos[4:, :]
    valid = [(posx >= float(k)).astype(f32) for k in range(5)]
    sp_sh = [spk[4 - k:M + 8 - k, :] for k in range(5)]
    nv = valid[0] + valid[1] + valid[2] + valid[3] + valid[4]
    c1 = sum(v * s for v, s in zip(valid, sp_sh))
    c0 = nv - c1
    inv_nv = 1.0 / nv
    inv_c0 = 1.0 / jnp.maximum(c0, 1.0)
    inv_c1 = 1.0 / jnp.maximum(c1, 1.0)
    x = root[4:, :]
    for k in range(5):
        co0 = valid[k] * ((1.0 - S) * inv_nv + S * (1.0 - sp_sh[k]) * inv_c0)
        co1 = valid[k] * S * sp_sh[k] * inv_c1
        x = x + co0 * h0[4 - k:M + 8 - k, :] + co1 * h1[4 - k:M + 8 - k, :]

    posn = pos[8:, :]
    agg = x[4:, :]
    for k in range(1, 5):
        agg = agg + (posn >= float(k)).astype(f32) * x[4 - k:M + 4 - k, :]

    xo = x[4:, :]
    x2 = (jnp.dot(agg, wgrel_ref[...], preferred_element_type=f32)
          + jnp.dot(xo, wgroot_ref[...], preferred_element_type=f32)
          + bgc_ref[...])
    skip = jnp.dot(xin[8:, :], wsk_ref[...], preferred_element_type=f32) + bsk_ref[...]
    out_ref[...] = jnp.dot(x2 + skip, wc_ref[...], preferred_element_type=f32) + bc_ref[...]


def kernel(input, dialog_lengths, speakers, W_rel, W_root, b_rgcn,
           Wgc_root, Wgc_rel, b_gc, W_skip, b_skip, W_cls, b_cls):
    N, D = input.shape
    n_cls = W_cls.shape[1]
    M = _M
    nblocks = (N + M - 1) // M
    Npad = nblocks * M
    tot = Npad + 8

    starts = jnp.cumsum(dialog_lengths) - dialog_lengths
    n_ids = jnp.arange(N, dtype=jnp.int32)
    start_n = jnp.max(jnp.where(starts[None, :] <= n_ids[:, None],
                                starts[None, :], 0), axis=1)
    pos = (n_ids - start_n).astype(jnp.float32)

    inp_p = jnp.zeros((tot, D), input.dtype).at[8:8 + N].set(input)
    spk_p = jnp.zeros((tot, 1), jnp.float32).at[8:8 + N, 0].set(
        speakers.astype(jnp.float32))
    pos_p = jnp.zeros((tot, 1), jnp.float32).at[8:8 + N, 0].set(pos)

    full = lambda shape: pl.BlockSpec(shape, lambda b: (0,) * len(shape))
    out = pl.pallas_call(
        _body,
        grid=(nblocks,),
        in_specs=[
            full((tot, D)), full((tot, 1)), full((tot, 1)),
            full((D, D)), full((D, D)), full((D, D)), full((1, D)),
            full((D, D)), full((D, D)), full((1, D)),
            full((D, D)), full((1, D)),
            full((D, n_cls)), full((1, n_cls)),
        ],
        out_specs=pl.BlockSpec((M, n_cls), lambda b: (b, 0)),
        out_shape=jax.ShapeDtypeStruct((Npad, n_cls), jnp.float32),
    )(inp_p, spk_p, pos_p, W_rel[0], W_rel[1], W_root,
      b_rgcn.reshape(1, D), Wgc_root, Wgc_rel, b_gc.reshape(1, D),
      W_skip, b_skip.reshape(1, D), W_cls, b_cls.reshape(1, n_cls))
    return out[:N]
